# baseline (speedup 1.0000x reference)
import functools

import jax
import jax.numpy as jnp
from jax import lax
from jax.experimental import pallas as pl
from jax.experimental.pallas import tpu as pltpu
from jax.experimental.pallas import tpu_sc as plsc

VOCAB = 100000
FACT = 64
HID = 768
STEP_NUM = 512
BEAT_RES = 4
BAR_STEP = 16
B = 1024
TOK = B * STEP_NUM
NPAIR = TOK // 2
EPS = 1e-8

_NC = 2
_NS = 16
_NW = _NC * _NS
_PER_W = NPAIR // _NW
_CH = 128
_NITER = _PER_W // _CH
_NBUF = 4


def _sc_gather_body(ids_hbm, table_hbm, out_hbm, idx_v, rows_v, gsem, wsem):
    wid = lax.axis_index("s") * _NC + lax.axis_index("c")
    base = wid * _PER_W
    pltpu.sync_copy(ids_hbm.at[pl.ds(base, _PER_W)], idx_v.at[0])
    pltpu.sync_copy(ids_hbm.at[pl.ds(NPAIR + base, _PER_W)], idx_v.at[1])

    @pl.loop(0, _NITER, step=_NBUF)
    def group(g):
        for b in range(_NBUF):
            for h in range(2):
                pltpu.make_async_copy(
                    table_hbm.at[idx_v.at[h, pl.ds((g + b) * _CH, _CH)]],
                    rows_v.at[b, h], gsem.at[b, h]).start()
        for b in range(_NBUF):
            for h in range(2):
                pltpu.make_async_copy(
                    table_hbm.at[idx_v.at[h, pl.ds((g + b) * _CH, _CH)]],
                    rows_v.at[b, h], gsem.at[b, h]).wait()
                pltpu.make_async_copy(
                    rows_v.at[b, h],
                    out_hbm.at[pl.ds(base + (g + b) * _CH, _CH),
                               pl.ds(h * FACT, FACT)],
                    wsem.at[b, h]).start()
        for b in range(_NBUF):
            for h in range(2):
                pltpu.make_async_copy(
                    rows_v.at[b, h],
                    out_hbm.at[pl.ds(base + (g + b) * _CH, _CH),
                               pl.ds(h * FACT, FACT)],
                    wsem.at[b, h]).wait()


def _sc_gather(ids_flat, table):
    mesh = plsc.VectorSubcoreMesh(core_axis_name="c", subcore_axis_name="s")
    f = functools.partial(
        pl.kernel,
        mesh=mesh,
        out_type=jax.ShapeDtypeStruct((NPAIR, 2 * FACT), jnp.float32),
        scratch_types=[
            pltpu.VMEM((2, _PER_W), jnp.int32),
            pltpu.VMEM((_NBUF, 2, _CH, FACT), jnp.float32),
            pltpu.SemaphoreType.DMA((_NBUF, 2)),
            pltpu.SemaphoreType.DMA((_NBUF, 2)),
        ],
        compiler_params=pltpu.CompilerParams(use_tc_tiling_on_sc=False),
    )(_sc_gather_body)
    return f(ids_flat, table)


_BB = 4


def _tc_body(g_ref, ct_ref, cw_ref, w_ref, gam_ref, bet_ref, out_ref, pos_s):
    @pl.when(pl.program_id(0) == 0)
    def _():
        pos_s[...] = jnp.dot(ct_ref[...], cw_ref[...],
                             preferred_element_type=jnp.float32)

    gp = g_ref[...]
    e = jnp.concatenate(
        [gp[..., :FACT].reshape(_BB * STEP_NUM, FACT),
         gp[..., FACT:].reshape(_BB * STEP_NUM, FACT)], axis=0)
    x = jnp.dot(e, w_ref[...], preferred_element_type=jnp.float32)
    x = x.reshape(2, _BB, STEP_NUM, HID) + pos_s[...][None, None, :, :]
    mu = jnp.mean(x, axis=-1, keepdims=True)
    xc = x - mu
    var = jnp.mean(xc * xc, axis=-1, keepdims=True)
    inv = 1.0 / jnp.sqrt(var + EPS)
    out_ref[...] = (xc * inv) * gam_ref[...] + bet_ref[...]


def _tc_main(g, cat_tbl, cat_W, input_W, gamma, beta):
    return pl.pallas_call(
        _tc_body,
        grid=((B // 2) // _BB,),
        in_specs=[
            pl.BlockSpec((_BB, STEP_NUM, 2 * FACT), lambda i: (i, 0, 0)),
            pl.BlockSpec(cat_tbl.shape, lambda i: (0, 0)),
            pl.BlockSpec(cat_W.shape, lambda i: (0, 0)),
            pl.BlockSpec(input_W.shape, lambda i: (0, 0)),
            pl.BlockSpec(gamma.shape, lambda i: (0, 0)),
            pl.BlockSpec(beta.shape, lambda i: (0, 0)),
        ],
        out_specs=pl.BlockSpec((2, _BB, STEP_NUM, HID),
                               lambda i: (0, i, 0, 0)),
        out_shape=jax.ShapeDtypeStruct((2, B // 2, STEP_NUM, HID),
                                       jnp.float32),
        scratch_shapes=[pltpu.VMEM((STEP_NUM, HID), jnp.float32)],
    )(g, cat_tbl, cat_W, input_W, gamma, beta)


def kernel(input_ids, input_table, input_W, step_table, step_W,
           beat_table, beat_W, bar_table, bar_W, gamma, beta):
    ids_flat = input_ids.reshape(TOK).astype(jnp.int32)
    cat_tbl = jnp.concatenate(
        [step_table,
         jnp.repeat(beat_table, BEAT_RES, axis=0),
         jnp.repeat(bar_table, BAR_STEP, axis=0)], axis=1)
    cat_W = jnp.concatenate([step_W, beat_W, bar_W], axis=0)

    g = _sc_gather(ids_flat, input_table)
    g = g.reshape(B // 2, STEP_NUM, 2 * FACT)
    out = _tc_main(g, cat_tbl, cat_W, input_W,
                   gamma.reshape(1, HID), beta.reshape(1, HID))
    return out.reshape(B, STEP_NUM, HID)


# speedup vs baseline: 1.0022x; 1.0022x over previous
import functools

import jax
import jax.numpy as jnp
from jax import lax
from jax.experimental import pallas as pl
from jax.experimental.pallas import tpu as pltpu
from jax.experimental.pallas import tpu_sc as plsc

VOCAB = 100000
FACT = 64
HID = 768
STEP_NUM = 512
BEAT_RES = 4
BAR_STEP = 16
B = 1024
TOK = B * STEP_NUM
NPAIR = TOK // 2
EPS = 1e-8

_NC = 2
_NS = 16
_NW = _NC * _NS
_NCHUNK = 2
NPC = NPAIR // _NCHUNK
_PER_W = NPC // _NW
_CH = 128
_NITER = _PER_W // _CH
_NBUF = 4


def _sc_gather_body(part, ids_hbm, table_hbm, out_hbm, idx_v, rows_v,
                    gsem, wsem):
    wid = lax.axis_index("s") * _NC + lax.axis_index("c")
    pbase = part * NPC
    base = wid * _PER_W
    pltpu.sync_copy(ids_hbm.at[pl.ds(pbase + base, _PER_W)], idx_v.at[0])
    pltpu.sync_copy(ids_hbm.at[pl.ds(NPAIR + pbase + base, _PER_W)],
                    idx_v.at[1])

    @pl.loop(0, _NITER, step=_NBUF)
    def group(g):
        for b in range(_NBUF):
            for h in range(2):
                pltpu.make_async_copy(
                    table_hbm.at[idx_v.at[h, pl.ds((g + b) * _CH, _CH)]],
                    rows_v.at[b, h], gsem.at[b, h]).start()
        for b in range(_NBUF):
            for h in range(2):
                pltpu.make_async_copy(
                    table_hbm.at[idx_v.at[h, pl.ds((g + b) * _CH, _CH)]],
                    rows_v.at[b, h], gsem.at[b, h]).wait()
                pltpu.make_async_copy(
                    rows_v.at[b, h],
                    out_hbm.at[pl.ds(base + (g + b) * _CH, _CH),
                               pl.ds(h * FACT, FACT)],
                    wsem.at[b, h]).start()
        for b in range(_NBUF):
            for h in range(2):
                pltpu.make_async_copy(
                    rows_v.at[b, h],
                    out_hbm.at[pl.ds(base + (g + b) * _CH, _CH),
                               pl.ds(h * FACT, FACT)],
                    wsem.at[b, h]).wait()


def _sc_gather(ids_flat, table, part):
    mesh = plsc.VectorSubcoreMesh(core_axis_name="c", subcore_axis_name="s")
    f = functools.partial(
        pl.kernel,
        mesh=mesh,
        out_type=jax.ShapeDtypeStruct((NPC, 2 * FACT), jnp.float32),
        scratch_types=[
            pltpu.VMEM((2, _PER_W), jnp.int32),
            pltpu.VMEM((_NBUF, 2, _CH, FACT), jnp.float32),
            pltpu.SemaphoreType.DMA((_NBUF, 2)),
            pltpu.SemaphoreType.DMA((_NBUF, 2)),
        ],
        compiler_params=pltpu.CompilerParams(use_tc_tiling_on_sc=False),
    )(functools.partial(_sc_gather_body, part))
    return f(ids_flat, table)


_BB = 4


def _tc_body(g_ref, ct_ref, cw_ref, w_ref, gam_ref, bet_ref, out_ref, pos_s):
    @pl.when(pl.program_id(0) == 0)
    def _():
        pos_s[...] = jnp.dot(ct_ref[...], cw_ref[...],
                             preferred_element_type=jnp.float32)

    gp = g_ref[...]
    e = jnp.concatenate(
        [gp[..., :FACT].reshape(_BB * STEP_NUM, FACT),
         gp[..., FACT:].reshape(_BB * STEP_NUM, FACT)], axis=0)
    x = jnp.dot(e, w_ref[...], preferred_element_type=jnp.float32)
    x = x.reshape(2, _BB, STEP_NUM, HID) + pos_s[...][None, None, :, :]
    mu = jnp.mean(x, axis=-1, keepdims=True)
    xc = x - mu
    var = jnp.mean(xc * xc, axis=-1, keepdims=True)
    inv = 1.0 / jnp.sqrt(var + EPS)
    out_ref[...] = (xc * inv) * gam_ref[...] + bet_ref[...]


def _tc_body_alias(prev_ref, g_ref, ct_ref, cw_ref, w_ref, gam_ref,
                   bet_ref, out_ref, pos_s):
    del prev_ref
    _tc_body(g_ref, ct_ref, cw_ref, w_ref, gam_ref, bet_ref, out_ref, pos_s)


def _tc_part(g, cat_tbl, cat_W, input_W, gamma, beta, part, prev=None):
    row0 = part * (B // 2 // _NCHUNK) // _BB
    common = dict(
        grid=((B // 2 // _NCHUNK) // _BB,),
        out_specs=pl.BlockSpec((2, _BB, STEP_NUM, HID),
                               lambda i: (0, row0 + i, 0, 0)),
        out_shape=jax.ShapeDtypeStruct((2, B // 2, STEP_NUM, HID),
                                       jnp.float32),
        scratch_shapes=[pltpu.VMEM((STEP_NUM, HID), jnp.float32)],
    )
    data_specs = [
        pl.BlockSpec((_BB, STEP_NUM, 2 * FACT), lambda i: (i, 0, 0)),
        pl.BlockSpec(cat_tbl.shape, lambda i: (0, 0)),
        pl.BlockSpec(cat_W.shape, lambda i: (0, 0)),
        pl.BlockSpec(input_W.shape, lambda i: (0, 0)),
        pl.BlockSpec(gamma.shape, lambda i: (0, 0)),
        pl.BlockSpec(beta.shape, lambda i: (0, 0)),
    ]
    if prev is None:
        return pl.pallas_call(
            _tc_body, in_specs=data_specs, **common,
        )(g, cat_tbl, cat_W, input_W, gamma, beta)
    return pl.pallas_call(
        _tc_body_alias,
        in_specs=[pl.BlockSpec(memory_space=pl.ANY)] + data_specs,
        input_output_aliases={0: 0},
        **common,
    )(prev, g, cat_tbl, cat_W, input_W, gamma, beta)


def kernel(input_ids, input_table, input_W, step_table, step_W,
           beat_table, beat_W, bar_table, bar_W, gamma, beta):
    ids_flat = input_ids.reshape(TOK).astype(jnp.int32)
    cat_tbl = jnp.concatenate(
        [step_table,
         jnp.repeat(beat_table, BEAT_RES, axis=0),
         jnp.repeat(bar_table, BAR_STEP, axis=0)], axis=1)
    cat_W = jnp.concatenate([step_W, beat_W, bar_W], axis=0)

    gamma2 = gamma.reshape(1, HID)
    beta2 = beta.reshape(1, HID)
    gs = [_sc_gather(ids_flat, input_table, part=p).reshape(
        B // 2 // _NCHUNK, STEP_NUM, 2 * FACT) for p in range(_NCHUNK)]
    out = None
    for p in range(_NCHUNK):
        out = _tc_part(gs[p], cat_tbl, cat_W, input_W, gamma2, beta2,
                       part=p, prev=out)
    return out.reshape(B, STEP_NUM, HID)


# speedup vs baseline: 1.0027x; 1.0005x over previous
import functools

import jax
import jax.numpy as jnp
from jax import lax
from jax.experimental import pallas as pl
from jax.experimental.pallas import tpu as pltpu
from jax.experimental.pallas import tpu_sc as plsc

VOCAB = 100000
FACT = 64
HID = 768
STEP_NUM = 512
BEAT_RES = 4
BAR_STEP = 16
B = 1024
TOK = B * STEP_NUM
NPAIR = TOK // 2
EPS = 1e-8

_NC = 2
_NS = 16
_NW = _NC * _NS
_NCHUNK = 2
NPC = NPAIR // _NCHUNK
_PER_W = NPC // _NW
_CH = 128
_NITER = _PER_W // _CH
_NBUF = 4


def _sc_gather_body(part, ids_hbm, table_hbm, out_hbm, idx_v, rows_v,
                    gsem, wsem):
    wid = lax.axis_index("s") * _NC + lax.axis_index("c")
    base = wid * _PER_W
    r_lo = part * (NPC // 4096) + wid
    r_hi = (B // 2) // 8 + r_lo
    pltpu.sync_copy(ids_hbm.at[r_lo], idx_v.at[0])
    pltpu.sync_copy(ids_hbm.at[r_hi], idx_v.at[1])

    @pl.loop(0, _NITER, step=_NBUF)
    def group(g):
        for b in range(_NBUF):
            m = g + b
            for h in range(2):
                pltpu.make_async_copy(
                    table_hbm.at[idx_v.at[h, m % 4, m // 4]],
                    rows_v.at[b, h], gsem.at[b, h]).start()
        for b in range(_NBUF):
            m = g + b
            for h in range(2):
                pltpu.make_async_copy(
                    table_hbm.at[idx_v.at[h, m % 4, m // 4]],
                    rows_v.at[b, h], gsem.at[b, h]).wait()
                pltpu.make_async_copy(
                    rows_v.at[b, h],
                    out_hbm.at[pl.ds(base + m * _CH, _CH),
                               pl.ds(h * FACT, FACT)],
                    wsem.at[b, h]).start()
        for b in range(_NBUF):
            m = g + b
            for h in range(2):
                pltpu.make_async_copy(
                    rows_v.at[b, h],
                    out_hbm.at[pl.ds(base + m * _CH, _CH),
                               pl.ds(h * FACT, FACT)],
                    wsem.at[b, h]).wait()


def _sc_gather(ids_flat, table, part):
    mesh = plsc.VectorSubcoreMesh(core_axis_name="c", subcore_axis_name="s")
    f = functools.partial(
        pl.kernel,
        mesh=mesh,
        out_type=jax.ShapeDtypeStruct((NPC, 2 * FACT), jnp.float32),
        scratch_types=[
            pltpu.VMEM((2, 4, 8, _CH), jnp.int32),
            pltpu.VMEM((_NBUF, 2, _CH, FACT), jnp.float32),
            pltpu.SemaphoreType.DMA((_NBUF, 2)),
            pltpu.SemaphoreType.DMA((_NBUF, 2)),
        ],
        compiler_params=pltpu.CompilerParams(use_tc_tiling_on_sc=False),
    )(functools.partial(_sc_gather_body, part))
    return f(ids_flat, table)


_BB = 4


def _tc_body(g_ref, ct_ref, cw_ref, w_ref, gam_ref, bet_ref, out_ref, pos_s):
    @pl.when(pl.program_id(0) == 0)
    def _():
        pos_s[...] = jnp.dot(ct_ref[...], cw_ref[...],
                             preferred_element_type=jnp.float32)

    gp = g_ref[...]
    e = jnp.concatenate(
        [gp[..., :FACT].reshape(_BB * STEP_NUM, FACT),
         gp[..., FACT:].reshape(_BB * STEP_NUM, FACT)], axis=0)
    x = jnp.dot(e, w_ref[...], preferred_element_type=jnp.float32)
    x = x.reshape(2, _BB, STEP_NUM, HID) + pos_s[...][None, None, :, :]
    mu = jnp.mean(x, axis=-1, keepdims=True)
    xc = x - mu
    var = jnp.mean(xc * xc, axis=-1, keepdims=True)
    inv = 1.0 / jnp.sqrt(var + EPS)
    out_ref[...] = (xc * inv) * gam_ref[...] + bet_ref[...]


def _tc_body_alias(prev_ref, g_ref, ct_ref, cw_ref, w_ref, gam_ref,
                   bet_ref, out_ref, pos_s):
    del prev_ref
    _tc_body(g_ref, ct_ref, cw_ref, w_ref, gam_ref, bet_ref, out_ref, pos_s)


def _tc_part(g, cat_tbl, cat_W, input_W, gamma, beta, part, prev=None):
    row0 = part * (B // 2 // _NCHUNK) // _BB
    common = dict(
        grid=((B // 2 // _NCHUNK) // _BB,),
        out_specs=pl.BlockSpec((2, _BB, STEP_NUM, HID),
                               lambda i: (0, row0 + i, 0, 0)),
        out_shape=jax.ShapeDtypeStruct((2, B // 2, STEP_NUM, HID),
                                       jnp.float32),
        scratch_shapes=[pltpu.VMEM((STEP_NUM, HID), jnp.float32)],
    )
    data_specs = [
        pl.BlockSpec((_BB, STEP_NUM, 2 * FACT), lambda i: (i, 0, 0)),
        pl.BlockSpec(cat_tbl.shape, lambda i: (0, 0)),
        pl.BlockSpec(cat_W.shape, lambda i: (0, 0)),
        pl.BlockSpec(input_W.shape, lambda i: (0, 0)),
        pl.BlockSpec(gamma.shape, lambda i: (0, 0)),
        pl.BlockSpec(beta.shape, lambda i: (0, 0)),
    ]
    if prev is None:
        return pl.pallas_call(
            _tc_body, in_specs=data_specs, **common,
        )(g, cat_tbl, cat_W, input_W, gamma, beta)
    return pl.pallas_call(
        _tc_body_alias,
        in_specs=[pl.BlockSpec(memory_space=pl.ANY)] + data_specs,
        input_output_aliases={0: 0},
        **common,
    )(prev, g, cat_tbl, cat_W, input_W, gamma, beta)


def kernel(input_ids, input_table, input_W, step_table, step_W,
           beat_table, beat_W, bar_table, bar_W, gamma, beta):
    ids4 = (input_ids.astype(jnp.int32)
            .reshape(128, 8, 4, 128).transpose(0, 2, 1, 3))
    cat_tbl = jnp.concatenate(
        [step_table,
         jnp.repeat(beat_table, BEAT_RES, axis=0),
         jnp.repeat(bar_table, BAR_STEP, axis=0)], axis=1)
    cat_W = jnp.concatenate([step_W, beat_W, bar_W], axis=0)

    gamma2 = gamma.reshape(1, HID)
    beta2 = beta.reshape(1, HID)
    gs = [_sc_gather(ids4, input_table, part=p).reshape(
        B // 2 // _NCHUNK, STEP_NUM, 2 * FACT) for p in range(_NCHUNK)]
    out = None
    for p in range(_NCHUNK):
        out = _tc_part(gs[p], cat_tbl, cat_W, input_W, gamma2, beta2,
                       part=p, prev=out)
    return out.reshape(B, STEP_NUM, HID)
